# baseline (speedup 1.0000x reference)
import functools

import jax
import jax.numpy as jnp
from jax import lax
from jax.experimental import pallas as pl

TOKENS = 16384
D_MODEL = 4096
N_EXPERTS = 64
BT = 1024
NK = 2
KS = D_MODEL // NK


def _router_block(*refs):
    x_refs = refs[:NK]
    w_ref = refs[NK]
    topi_ref, topv_ref = refs[NK + 1], refs[NK + 2]
    scores = jnp.zeros((BT, N_EXPERTS), jnp.float32)
    for j in range(NK):
        scores += lax.dot_general(
            x_refs[j][...], w_ref[:, j * KS:(j + 1) * KS],
            dimension_numbers=(((1,), (1,)), ((), ())),
            preferred_element_type=jnp.float32,
        )
    iota = lax.broadcasted_iota(jnp.int32, scores.shape, 1)
    m1 = jnp.max(scores, axis=1, keepdims=True)
    i1 = jnp.min(jnp.where(scores == m1, iota, N_EXPERTS), axis=1, keepdims=True)
    masked = jnp.where(iota == i1, -jnp.inf, scores)
    m2 = jnp.max(masked, axis=1, keepdims=True)
    i2 = jnp.min(jnp.where(masked == m2, iota, N_EXPERTS), axis=1, keepdims=True)
    e2 = jnp.exp(m2 - m1)
    p1 = 1.0 / (1.0 + e2)
    p2 = e2 / (1.0 + e2)
    topi_ref[...] = jnp.concatenate([i1, i2], axis=1)
    topv_ref[...] = jnp.concatenate([p1, p2], axis=1)


@jax.jit
def kernel(x, W):
    grid = (TOKENS // BT,)
    topi, topv = pl.pallas_call(
        _router_block,
        grid=grid,
        in_specs=[
            pl.BlockSpec((BT, KS), functools.partial(lambda j, t: (t, j), j))
            for j in range(NK)
        ] + [
            pl.BlockSpec((N_EXPERTS, D_MODEL), lambda t: (0, 0)),
        ],
        out_specs=[
            pl.BlockSpec((BT, 2), lambda t: (t, 0)),
            pl.BlockSpec((BT, 2), lambda t: (t, 0)),
        ],
        out_shape=[
            jax.ShapeDtypeStruct((TOKENS, 2), jnp.int32),
            jax.ShapeDtypeStruct((TOKENS, 2), jnp.float32),
        ],
    )(*([x] * NK), W)
    return (topi, topv)


# speedup vs baseline: 1.0013x; 1.0013x over previous
import functools

import jax
import jax.numpy as jnp
from jax import lax
from jax.experimental import pallas as pl

TOKENS = 16384
D_MODEL = 4096
N_EXPERTS = 64
BT = 1024
NK = 4
KS = D_MODEL // NK


def _router_block(*refs):
    x_refs = refs[:NK]
    w_ref = refs[NK]
    topi_ref, topv_ref = refs[NK + 1], refs[NK + 2]
    scores = jnp.zeros((BT, N_EXPERTS), jnp.float32)
    for j in range(NK):
        scores += lax.dot_general(
            x_refs[j][...], w_ref[:, j * KS:(j + 1) * KS],
            dimension_numbers=(((1,), (1,)), ((), ())),
            preferred_element_type=jnp.float32,
        )
    iota = lax.broadcasted_iota(jnp.int32, scores.shape, 1)
    m1 = jnp.max(scores, axis=1, keepdims=True)
    i1 = jnp.min(jnp.where(scores == m1, iota, N_EXPERTS), axis=1, keepdims=True)
    masked = jnp.where(iota == i1, -jnp.inf, scores)
    m2 = jnp.max(masked, axis=1, keepdims=True)
    i2 = jnp.min(jnp.where(masked == m2, iota, N_EXPERTS), axis=1, keepdims=True)
    e2 = jnp.exp(m2 - m1)
    p1 = 1.0 / (1.0 + e2)
    p2 = e2 / (1.0 + e2)
    topi_ref[...] = jnp.concatenate([i1, i2], axis=1)
    topv_ref[...] = jnp.concatenate([p1, p2], axis=1)


@jax.jit
def kernel(x, W):
    grid = (TOKENS // BT,)
    topi, topv = pl.pallas_call(
        _router_block,
        grid=grid,
        in_specs=[
            pl.BlockSpec((BT, KS), functools.partial(lambda j, t: (t, j), j))
            for j in range(NK)
        ] + [
            pl.BlockSpec((N_EXPERTS, D_MODEL), lambda t: (0, 0)),
        ],
        out_specs=[
            pl.BlockSpec((BT, 2), lambda t: (t, 0)),
            pl.BlockSpec((BT, 2), lambda t: (t, 0)),
        ],
        out_shape=[
            jax.ShapeDtypeStruct((TOKENS, 2), jnp.int32),
            jax.ShapeDtypeStruct((TOKENS, 2), jnp.float32),
        ],
    )(*([x] * NK), W)
    return (topi, topv)
